# baseline (speedup 1.0000x reference)
import functools

import jax
import jax.numpy as jnp
from jax import lax
from jax.experimental import pallas as pl
from jax.experimental.pallas import tpu as pltpu
from jax.experimental.pallas import tpu_sc as plsc

T, D, F, E = 2048, 1024, 2048, 8
K = 2
BLK = 512
R = 16
P = R * BLK
FC = 512

_NC = 2
_NS = 16
_NW = _NC * _NS


def _router_body(x_ref, wr_ref, idx_ref, w_ref):
    x = x_ref[...]
    wr = wr_ref[...]
    logits = jnp.dot(x, wr, preferred_element_type=jnp.float32)
    m = jnp.max(logits, axis=1, keepdims=True)
    ex = jnp.exp(logits - m)
    p = ex / jnp.sum(ex, axis=1, keepdims=True)
    iota = lax.broadcasted_iota(jnp.int32, p.shape, 1)
    m1 = jnp.max(p, axis=1, keepdims=True)
    i1 = jnp.min(jnp.where(p == m1, iota, E), axis=1, keepdims=True)
    p2 = jnp.where(iota == i1, -jnp.inf, p)
    m2 = jnp.max(p2, axis=1, keepdims=True)
    i2 = jnp.min(jnp.where(p2 == m2, iota, E), axis=1, keepdims=True)
    t = jnp.exp(m2 - m1)
    w0 = 1.0 / (1.0 + t)
    idx_ref[...] = jnp.where(iota == 0, i1, jnp.where(iota == 1, i2, 0))
    w_ref[...] = jnp.where(iota == 0, w0, jnp.where(iota == 1, 1.0 - w0, 0.0))


def _route(x2d, W_router):
    return pl.pallas_call(
        _router_body,
        out_shape=[jax.ShapeDtypeStruct((T, E), jnp.int32),
                   jax.ShapeDtypeStruct((T, E), jnp.float32)],
    )(x2d, W_router)


_GCH = 32
_GNCH = (P // _NW) // _GCH


def _gather_body(x_hbm, idx_hbm, out_hbm, idx_v, b0, b1, g0, g1, s0, s1):
    wid = lax.axis_index("s") * _NC + lax.axis_index("c")
    rows_per_w = P // _NW
    base = wid * rows_per_w
    pltpu.sync_copy(idx_hbm.at[pl.ds(base, rows_per_w)], idx_v)
    bufs, gsems, ssems = (b0, b1), (g0, g1), (s0, s1)
    hg = [None] * _GNCH
    hs = [None] * _GNCH
    hg[0] = pltpu.async_copy(x_hbm.at[idx_v.at[pl.ds(0, _GCH)]], b0, g0)
    for c in range(_GNCH):
        cur = c % 2
        hg[c].wait()
        if c + 1 < _GNCH:
            if c - 1 >= 0:
                hs[c - 1].wait()
            nxt = (c + 1) % 2
            hg[c + 1] = pltpu.async_copy(
                x_hbm.at[idx_v.at[pl.ds((c + 1) * _GCH, _GCH)]],
                bufs[nxt], gsems[nxt])
        hs[c] = pltpu.async_copy(
            bufs[cur], out_hbm.at[pl.ds(base + c * _GCH, _GCH)], ssems[cur])
    hs[_GNCH - 1].wait()
    hs[_GNCH - 2].wait()


def _sc_gather(x2d, src_token):
    k = pl.kernel(
        _gather_body,
        mesh=plsc.VectorSubcoreMesh(core_axis_name="c", subcore_axis_name="s"),
        out_type=jax.ShapeDtypeStruct((P, D), jnp.float32),
        scratch_types=[pltpu.VMEM((P // _NW,), jnp.int32),
                       pltpu.VMEM((_GCH, D), jnp.float32),
                       pltpu.VMEM((_GCH, D), jnp.float32),
                       pltpu.SemaphoreType.DMA,
                       pltpu.SemaphoreType.DMA,
                       pltpu.SemaphoreType.DMA,
                       pltpu.SemaphoreType.DMA],
    )
    return k(x2d, src_token)


def _gemm_body(te_ref, tv_ref, x_ref, wg_ref, wu_ref, wd_ref, ws_ref, y_ref):
    r = pl.program_id(0)
    valid = tv_ref[r] > 0

    @pl.when(valid)
    def _():
        xb = x_ref[...]
        acc = jnp.zeros((BLK, D), jnp.float32)
        for fc in range(F // FC):
            wg = wg_ref[0, :, fc * FC:(fc + 1) * FC]
            wu = wu_ref[0, :, fc * FC:(fc + 1) * FC]
            g = jnp.dot(xb, wg, preferred_element_type=jnp.float32)
            u = jnp.dot(xb, wu, preferred_element_type=jnp.float32)
            h = (g * jax.nn.sigmoid(g)) * u
            wd = wd_ref[0, fc * FC:(fc + 1) * FC, :]
            acc = acc + jnp.dot(h, wd, preferred_element_type=jnp.float32)
        w = ws_ref[0, 0, :]
        y_ref[...] = acc * w[:, None]

    @pl.when(jnp.logical_not(valid))
    def _():
        y_ref[...] = jnp.zeros((BLK, D), jnp.float32)


def _grouped_gemm(tile_expert, tile_valid, xs, Wg, Wu, Wd, ws3):
    grid_spec = pltpu.PrefetchScalarGridSpec(
        num_scalar_prefetch=2,
        grid=(R,),
        in_specs=[
            pl.BlockSpec((BLK, D), lambda r, te, tv: (r, 0)),
            pl.BlockSpec((1, D, F), lambda r, te, tv: (te[r], 0, 0)),
            pl.BlockSpec((1, D, F), lambda r, te, tv: (te[r], 0, 0)),
            pl.BlockSpec((1, F, D), lambda r, te, tv: (te[r], 0, 0)),
            pl.BlockSpec((1, 1, BLK), lambda r, te, tv: (r, 0, 0)),
        ],
        out_specs=pl.BlockSpec((BLK, D), lambda r, te, tv: (r, 0)),
    )
    return pl.pallas_call(
        _gemm_body,
        grid_spec=grid_spec,
        out_shape=jax.ShapeDtypeStruct((P, D), jnp.float32),
        compiler_params=pltpu.CompilerParams(
            vmem_limit_bytes=100 * 1024 * 1024),
    )(tile_expert, tile_valid, xs, Wg, Wu, Wd, ws3)


def _combine_body(ys_hbm, g0_hbm, g1_hbm, out_hbm, i0_v, i1_v, r0_v, r1_v, sem):
    wid = lax.axis_index("s") * _NC + lax.axis_index("c")
    toks_per_w = T // _NW
    ch = 32
    nvec = ch * D // 16
    for c in range(toks_per_w // ch):
        off = wid * toks_per_w + c * ch
        pltpu.sync_copy(g0_hbm.at[pl.ds(off, ch)], i0_v)
        pltpu.sync_copy(g1_hbm.at[pl.ds(off, ch)], i1_v)
        pltpu.async_copy(ys_hbm.at[i0_v], r0_v, sem).wait()
        pltpu.async_copy(ys_hbm.at[i1_v], r1_v, sem).wait()

        def body(i, _):
            row = i // (D // 16)
            col = (i % (D // 16)) * 16
            r0_v[row, pl.ds(col, 16)] = (r0_v[row, pl.ds(col, 16)]
                                         + r1_v[row, pl.ds(col, 16)])
            return 0

        lax.fori_loop(0, nvec, body, 0)
        pltpu.sync_copy(r0_v, out_hbm.at[pl.ds(off, ch)])


def _sc_combine(ys, g0, g1):
    k = pl.kernel(
        _combine_body,
        mesh=plsc.VectorSubcoreMesh(core_axis_name="c", subcore_axis_name="s"),
        out_type=jax.ShapeDtypeStruct((T, D), jnp.float32),
        scratch_types=[pltpu.VMEM((32,), jnp.int32),
                       pltpu.VMEM((32,), jnp.int32),
                       pltpu.VMEM((32, D), jnp.float32),
                       pltpu.VMEM((32, D), jnp.float32),
                       pltpu.SemaphoreType.DMA],
    )
    return k(ys, g0, g1)


def _dispatch_meta(sel, wts):
    e_flat = sel.reshape(-1)
    oh = (e_flat[:, None] == jnp.arange(E)[None, :]).astype(jnp.int32)
    ranks_full = jnp.cumsum(oh, axis=0) - oh
    rank = jnp.sum(ranks_full * oh, axis=1)
    counts = jnp.sum(oh, axis=0)
    blocks = (counts + BLK - 1) // BLK
    cum_blocks = jnp.cumsum(blocks)
    astart = (jnp.concatenate([jnp.zeros((1,), jnp.int32),
                               cum_blocks[:-1].astype(jnp.int32)]) * BLK)
    slot = astart[e_flat] + rank
    total_blocks = cum_blocks[-1]
    r_iota = jnp.arange(R)
    te_raw = jnp.searchsorted(cum_blocks, r_iota, side="right")
    last_te = te_raw[jnp.maximum(total_blocks - 1, 0)]
    tile_expert = jnp.where(r_iota < total_blocks, te_raw, last_te)
    tile_expert = jnp.minimum(tile_expert, E - 1).astype(jnp.int32)
    tile_valid = (r_iota < total_blocks).astype(jnp.int32)
    tok_ids = (jnp.arange(T * K, dtype=jnp.int32) // K)
    src_token = jnp.zeros((P,), jnp.int32).at[slot].set(tok_ids)
    ws = jnp.zeros((P,), jnp.float32).at[slot].set(wts.reshape(-1))
    return slot, src_token, ws, tile_expert, tile_valid


def kernel(x, W_router, Wg, Wu, Wd):
    x2d = x.reshape(T, D).astype(jnp.float32)
    idx8, w8 = _route(x2d, W_router)
    sel = idx8[:, :K]
    wts = w8[:, :K]
    slot, src_token, ws, tile_expert, tile_valid = _dispatch_meta(sel, wts)
    xs = _sc_gather(x2d, src_token)
    ys = _grouped_gemm(tile_expert, tile_valid, xs, Wg, Wu, Wd,
                       ws.reshape(R, 1, BLK))
    slot2 = slot.reshape(T, K)
    out2d = _sc_combine(ys, slot2[:, 0], slot2[:, 1])
    return out2d.reshape(x.shape)


# speedup vs baseline: 1.0162x; 1.0162x over previous
import functools

import jax
import jax.numpy as jnp
from jax import lax
from jax.experimental import pallas as pl
from jax.experimental.pallas import tpu as pltpu
from jax.experimental.pallas import tpu_sc as plsc

T, D, F, E = 2048, 1024, 2048, 8
K = 2
BLK = 512
R = 16
P = R * BLK
FC = 512

_NC = 2
_NS = 16
_NW = _NC * _NS


def _router_body(x_ref, wr_ref, idx_ref, w_ref):
    x = x_ref[...]
    wr = wr_ref[...]
    logits = jnp.dot(x, wr, preferred_element_type=jnp.float32)
    m = jnp.max(logits, axis=1, keepdims=True)
    ex = jnp.exp(logits - m)
    p = ex / jnp.sum(ex, axis=1, keepdims=True)
    iota = lax.broadcasted_iota(jnp.int32, p.shape, 1)
    m1 = jnp.max(p, axis=1, keepdims=True)
    i1 = jnp.min(jnp.where(p == m1, iota, E), axis=1, keepdims=True)
    p2 = jnp.where(iota == i1, -jnp.inf, p)
    m2 = jnp.max(p2, axis=1, keepdims=True)
    i2 = jnp.min(jnp.where(p2 == m2, iota, E), axis=1, keepdims=True)
    t = jnp.exp(m2 - m1)
    w0 = 1.0 / (1.0 + t)
    idx_ref[...] = jnp.where(iota == 0, i1, jnp.where(iota == 1, i2, 0))
    w_ref[...] = jnp.where(iota == 0, w0, jnp.where(iota == 1, 1.0 - w0, 0.0))


def _route(x2d, W_router):
    return pl.pallas_call(
        _router_body,
        out_shape=[jax.ShapeDtypeStruct((T, E), jnp.int32),
                   jax.ShapeDtypeStruct((T, E), jnp.float32)],
    )(x2d, W_router)


_GCH = 32


def _make_gather(n_rows):
    rows_per_w = n_rows // _NW
    nch = rows_per_w // _GCH

    def body(src_hbm, idx_hbm, out_hbm, idx_v, b0, b1, g0, g1, s0, s1):
        wid = lax.axis_index("s") * _NC + lax.axis_index("c")
        base = wid * rows_per_w
        pltpu.sync_copy(idx_hbm.at[pl.ds(base, rows_per_w)], idx_v)
        bufs, gsems, ssems = (b0, b1), (g0, g1), (s0, s1)
        hg = [None] * nch
        hs = [None] * nch
        hg[0] = pltpu.async_copy(src_hbm.at[idx_v.at[pl.ds(0, _GCH)]], b0, g0)
        for c in range(nch):
            cur = c % 2
            hg[c].wait()
            if c + 1 < nch:
                if c - 1 >= 0:
                    hs[c - 1].wait()
                nxt = (c + 1) % 2
                hg[c + 1] = pltpu.async_copy(
                    src_hbm.at[idx_v.at[pl.ds((c + 1) * _GCH, _GCH)]],
                    bufs[nxt], gsems[nxt])
            hs[c] = pltpu.async_copy(
                bufs[cur], out_hbm.at[pl.ds(base + c * _GCH, _GCH)],
                ssems[cur])
        hs[nch - 1].wait()
        if nch >= 2:
            hs[nch - 2].wait()

    def run(src, idx):
        k = pl.kernel(
            body,
            mesh=plsc.VectorSubcoreMesh(core_axis_name="c",
                                        subcore_axis_name="s"),
            out_type=jax.ShapeDtypeStruct((n_rows, D), jnp.float32),
            scratch_types=[pltpu.VMEM((rows_per_w,), jnp.int32),
                           pltpu.VMEM((_GCH, D), jnp.float32),
                           pltpu.VMEM((_GCH, D), jnp.float32),
                           pltpu.SemaphoreType.DMA,
                           pltpu.SemaphoreType.DMA,
                           pltpu.SemaphoreType.DMA,
                           pltpu.SemaphoreType.DMA],
        )
        return k(src, idx)

    return run


def _gemm_body(te_ref, tv_ref, x_ref, wg_ref, wu_ref, wd_ref, ws_ref, y_ref):
    r = pl.program_id(0)
    valid = tv_ref[r] > 0

    @pl.when(valid)
    def _():
        xb = x_ref[...]
        acc = jnp.zeros((BLK, D), jnp.float32)
        for fc in range(F // FC):
            wg = wg_ref[0, :, fc * FC:(fc + 1) * FC]
            wu = wu_ref[0, :, fc * FC:(fc + 1) * FC]
            g = jnp.dot(xb, wg, preferred_element_type=jnp.float32)
            u = jnp.dot(xb, wu, preferred_element_type=jnp.float32)
            h = (g * jax.nn.sigmoid(g)) * u
            wd = wd_ref[0, fc * FC:(fc + 1) * FC, :]
            acc = acc + jnp.dot(h, wd, preferred_element_type=jnp.float32)
        w = ws_ref[0, 0, :]
        y_ref[...] = acc * w[:, None]

    @pl.when(jnp.logical_not(valid))
    def _():
        y_ref[...] = jnp.zeros((BLK, D), jnp.float32)


def _grouped_gemm(tile_expert, tile_valid, xs, Wg, Wu, Wd, ws3):
    grid_spec = pltpu.PrefetchScalarGridSpec(
        num_scalar_prefetch=2,
        grid=(R,),
        in_specs=[
            pl.BlockSpec((BLK, D), lambda r, te, tv: (r, 0)),
            pl.BlockSpec((1, D, F), lambda r, te, tv: (te[r], 0, 0)),
            pl.BlockSpec((1, D, F), lambda r, te, tv: (te[r], 0, 0)),
            pl.BlockSpec((1, F, D), lambda r, te, tv: (te[r], 0, 0)),
            pl.BlockSpec((1, 1, BLK), lambda r, te, tv: (r, 0, 0)),
        ],
        out_specs=pl.BlockSpec((BLK, D), lambda r, te, tv: (r, 0)),
    )
    return pl.pallas_call(
        _gemm_body,
        grid_spec=grid_spec,
        out_shape=jax.ShapeDtypeStruct((P, D), jnp.float32),
        compiler_params=pltpu.CompilerParams(
            vmem_limit_bytes=100 * 1024 * 1024),
    )(tile_expert, tile_valid, xs, Wg, Wu, Wd, ws3)


_AB = 256


def _add_body(a_ref, b_ref, o_ref):
    o_ref[...] = a_ref[...] + b_ref[...]


def _tc_pair_add(ysg):
    nb = T // _AB
    return pl.pallas_call(
        _add_body,
        grid=(nb,),
        in_specs=[pl.BlockSpec((_AB, D), lambda r: (r, 0)),
                  pl.BlockSpec((_AB, D), lambda r: (r + T // _AB, 0))],
        out_specs=pl.BlockSpec((_AB, D), lambda r: (r, 0)),
        out_shape=jax.ShapeDtypeStruct((T, D), jnp.float32),
    )(ysg, ysg)


def _dispatch_meta(sel, wts):
    e_flat = sel.reshape(-1)
    oh = (e_flat[:, None] == jnp.arange(E)[None, :]).astype(jnp.int32)
    ranks_full = jnp.cumsum(oh, axis=0) - oh
    rank = jnp.sum(ranks_full * oh, axis=1)
    counts = jnp.sum(oh, axis=0)
    blocks = (counts + BLK - 1) // BLK
    cum_blocks = jnp.cumsum(blocks)
    astart = (jnp.concatenate([jnp.zeros((1,), jnp.int32),
                               cum_blocks[:-1].astype(jnp.int32)]) * BLK)
    slot = astart[e_flat] + rank
    total_blocks = cum_blocks[-1]
    r_iota = jnp.arange(R)
    te_raw = jnp.searchsorted(cum_blocks, r_iota, side="right")
    last_te = te_raw[jnp.maximum(total_blocks - 1, 0)]
    tile_expert = jnp.where(r_iota < total_blocks, te_raw, last_te)
    tile_expert = jnp.minimum(tile_expert, E - 1).astype(jnp.int32)
    tile_valid = (r_iota < total_blocks).astype(jnp.int32)
    tok_ids = (jnp.arange(T * K, dtype=jnp.int32) // K)
    src_token = jnp.zeros((P,), jnp.int32).at[slot].set(tok_ids)
    ws = jnp.zeros((P,), jnp.float32).at[slot].set(wts.reshape(-1))
    return slot, src_token, ws, tile_expert, tile_valid


def kernel(x, W_router, Wg, Wu, Wd):
    x2d = x.reshape(T, D).astype(jnp.float32)
    idx8, w8 = _route(x2d, W_router)
    sel = idx8[:, :K]
    wts = w8[:, :K]
    slot, src_token, ws, tile_expert, tile_valid = _dispatch_meta(sel, wts)
    xs = _make_gather(P)(x2d, src_token)
    ys = _grouped_gemm(tile_expert, tile_valid, xs, Wg, Wu, Wd,
                       ws.reshape(R, 1, BLK))
    slot2 = slot.reshape(T, K)
    gidx = jnp.concatenate([slot2[:, 0], slot2[:, 1]])
    ysg = _make_gather(2 * T)(ys, gidx)
    out2d = _tc_pair_add(ysg)
    return out2d.reshape(x.shape)


# speedup vs baseline: 1.8505x; 1.8210x over previous
import functools

import jax
import jax.numpy as jnp
from jax import lax
from jax.experimental import pallas as pl
from jax.experimental.pallas import tpu as pltpu
from jax.experimental.pallas import tpu_sc as plsc

T, D, F, E = 2048, 1024, 2048, 8
K = 2
BLK = 512
R = 16
P = R * BLK
FC = 512

_NC = 2
_NS = 16
_NW = _NC * _NS


def _router_body(x_ref, wr_ref, idx_ref, w_ref):
    x = x_ref[...]
    wr = wr_ref[...]
    logits = jnp.dot(x, wr, preferred_element_type=jnp.float32)
    m = jnp.max(logits, axis=1, keepdims=True)
    ex = jnp.exp(logits - m)
    p = ex / jnp.sum(ex, axis=1, keepdims=True)
    iota = lax.broadcasted_iota(jnp.int32, p.shape, 1)
    m1 = jnp.max(p, axis=1, keepdims=True)
    i1 = jnp.min(jnp.where(p == m1, iota, E), axis=1, keepdims=True)
    p2 = jnp.where(iota == i1, -jnp.inf, p)
    m2 = jnp.max(p2, axis=1, keepdims=True)
    i2 = jnp.min(jnp.where(p2 == m2, iota, E), axis=1, keepdims=True)
    t = jnp.exp(m2 - m1)
    w0 = 1.0 / (1.0 + t)
    idx_ref[...] = jnp.where(iota == 0, i1, jnp.where(iota == 1, i2, 0))
    w_ref[...] = jnp.where(iota == 0, w0, jnp.where(iota == 1, 1.0 - w0, 0.0))


def _route(x2d, W_router):
    return pl.pallas_call(
        _router_body,
        out_shape=[jax.ShapeDtypeStruct((T, E), jnp.int32),
                   jax.ShapeDtypeStruct((T, E), jnp.float32)],
    )(x2d, W_router)


_GCH = 32


def _make_gather(n_rows):
    rows_per_w = n_rows // _NW
    nch = rows_per_w // _GCH

    def body(src_hbm, idx_hbm, out_hbm, idx_v, b0, b1, g0, g1, s0, s1):
        wid = lax.axis_index("s") * _NC + lax.axis_index("c")
        base = wid * rows_per_w
        pltpu.sync_copy(idx_hbm.at[pl.ds(base, rows_per_w)], idx_v)
        bufs, gsems, ssems = (b0, b1), (g0, g1), (s0, s1)
        hg = [None] * nch
        hs = [None] * nch
        hg[0] = pltpu.async_copy(src_hbm.at[idx_v.at[pl.ds(0, _GCH)]], b0, g0)
        for c in range(nch):
            cur = c % 2
            hg[c].wait()
            if c + 1 < nch:
                if c - 1 >= 0:
                    hs[c - 1].wait()
                nxt = (c + 1) % 2
                hg[c + 1] = pltpu.async_copy(
                    src_hbm.at[idx_v.at[pl.ds((c + 1) * _GCH, _GCH)]],
                    bufs[nxt], gsems[nxt])
            hs[c] = pltpu.async_copy(
                bufs[cur], out_hbm.at[pl.ds(base + c * _GCH, _GCH)],
                ssems[cur])
        hs[nch - 1].wait()
        if nch >= 2:
            hs[nch - 2].wait()

    def run(src, idx):
        k = pl.kernel(
            body,
            mesh=plsc.VectorSubcoreMesh(core_axis_name="c",
                                        subcore_axis_name="s"),
            out_type=jax.ShapeDtypeStruct((n_rows, D), jnp.float32),
            scratch_types=[pltpu.VMEM((rows_per_w,), jnp.int32),
                           pltpu.VMEM((_GCH, D), jnp.float32),
                           pltpu.VMEM((_GCH, D), jnp.float32),
                           pltpu.SemaphoreType.DMA,
                           pltpu.SemaphoreType.DMA,
                           pltpu.SemaphoreType.DMA,
                           pltpu.SemaphoreType.DMA],
        )
        return k(src, idx)

    return run


def _gemm_body(te_ref, tv_ref, x_ref, wg_ref, wu_ref, wd_ref, ws_ref, y_ref):
    r = pl.program_id(0)
    valid = tv_ref[r] > 0

    @pl.when(valid)
    def _():
        xb = x_ref[...]
        acc = jnp.zeros((BLK, D), jnp.float32)
        for fc in range(F // FC):
            wg = wg_ref[0, :, fc * FC:(fc + 1) * FC]
            wu = wu_ref[0, :, fc * FC:(fc + 1) * FC]
            g = jnp.dot(xb, wg, preferred_element_type=jnp.float32)
            u = jnp.dot(xb, wu, preferred_element_type=jnp.float32)
            h = (g * jax.nn.sigmoid(g)) * u
            wd = wd_ref[0, fc * FC:(fc + 1) * FC, :]
            acc = acc + jnp.dot(h, wd, preferred_element_type=jnp.float32)
        w = ws_ref[0, 0, :]
        y_ref[...] = acc * w[:, None]

    @pl.when(jnp.logical_not(valid))
    def _():
        y_ref[...] = jnp.zeros((BLK, D), jnp.float32)


def _grouped_gemm(tile_expert, tile_valid, xs, Wg, Wu, Wd, ws3):
    grid_spec = pltpu.PrefetchScalarGridSpec(
        num_scalar_prefetch=2,
        grid=(R,),
        in_specs=[
            pl.BlockSpec((BLK, D), lambda r, te, tv: (r, 0)),
            pl.BlockSpec((1, D, F), lambda r, te, tv: (te[r], 0, 0)),
            pl.BlockSpec((1, D, F), lambda r, te, tv: (te[r], 0, 0)),
            pl.BlockSpec((1, F, D), lambda r, te, tv: (te[r], 0, 0)),
            pl.BlockSpec((1, 1, BLK), lambda r, te, tv: (r, 0, 0)),
        ],
        out_specs=pl.BlockSpec((BLK, D), lambda r, te, tv: (r, 0)),
    )
    return pl.pallas_call(
        _gemm_body,
        grid_spec=grid_spec,
        out_shape=jax.ShapeDtypeStruct((P, D), jnp.float32),
        compiler_params=pltpu.CompilerParams(
            vmem_limit_bytes=100 * 1024 * 1024),
    )(tile_expert, tile_valid, xs, Wg, Wu, Wd, ws3)


_AB = 256


def _add_body(a_ref, b_ref, o_ref):
    o_ref[...] = a_ref[...] + b_ref[...]


def _tc_pair_add(ysg):
    nb = T // _AB
    return pl.pallas_call(
        _add_body,
        grid=(nb,),
        in_specs=[pl.BlockSpec((_AB, D), lambda r: (r, 0)),
                  pl.BlockSpec((_AB, D), lambda r: (r + T // _AB, 0))],
        out_specs=pl.BlockSpec((_AB, D), lambda r: (r, 0)),
        out_shape=jax.ShapeDtypeStruct((T, D), jnp.float32),
    )(ysg, ysg)


def _dispatch_meta(sel, wts):
    e_flat = sel.reshape(-1)
    oh = (e_flat[:, None] == jnp.arange(E)[None, :]).astype(jnp.int32)
    ranks_full = jnp.cumsum(oh, axis=0) - oh
    rank = jnp.sum(ranks_full * oh, axis=1)
    counts = jnp.sum(oh, axis=0)
    blocks = (counts + BLK - 1) // BLK
    cum_blocks = jnp.cumsum(blocks)
    astart = (jnp.concatenate([jnp.zeros((1,), jnp.int32),
                               cum_blocks[:-1].astype(jnp.int32)]) * BLK)
    slot = astart[e_flat] + rank
    total_blocks = cum_blocks[-1]
    r_iota = jnp.arange(R)
    te_raw = jnp.searchsorted(cum_blocks, r_iota, side="right")
    last_te = te_raw[jnp.maximum(total_blocks - 1, 0)]
    tile_expert = jnp.where(r_iota < total_blocks, te_raw, last_te)
    tile_expert = jnp.minimum(tile_expert, E - 1).astype(jnp.int32)
    tile_valid = (r_iota < total_blocks).astype(jnp.int32)
    tok_ids = (jnp.arange(T * K, dtype=jnp.int32) // K)
    src_token = (jnp.arange(P, dtype=jnp.int32) % T).at[slot].set(tok_ids)
    ws = jnp.zeros((P,), jnp.float32).at[slot].set(wts.reshape(-1))
    return slot, src_token, ws, tile_expert, tile_valid


def kernel(x, W_router, Wg, Wu, Wd):
    x2d = x.reshape(T, D).astype(jnp.float32)
    idx8, w8 = _route(x2d, W_router)
    sel = idx8[:, :K]
    wts = w8[:, :K]
    slot, src_token, ws, tile_expert, tile_valid = _dispatch_meta(sel, wts)
    xs = _make_gather(P)(x2d, src_token)
    ys = _grouped_gemm(tile_expert, tile_valid, xs, Wg, Wu, Wd,
                       ws.reshape(R, 1, BLK))
    slot2 = slot.reshape(T, K)
    gidx = jnp.concatenate([slot2[:, 0], slot2[:, 1]])
    ysg = _make_gather(2 * T)(ys, gidx)
    out2d = _tc_pair_add(ysg)
    return out2d.reshape(x.shape)
